# baseline (speedup 1.0000x reference)
import functools

import jax
import jax.numpy as jnp
from jax import lax
from jax.experimental import pallas as pl
from jax.experimental.pallas import tpu as pltpu
from jax.experimental.pallas import tpu_sc as plsc

_EMB_D = 768
_E_POOL = 100
_E_P_LEN = 40
_B = 256
_HALF = _E_P_LEN // 2
_ROW = _HALF * _EMB_D
_NC = 2
_NS = 16
_NW = _NC * _NS
_BPW = _B // _NW


def _route_body(xq_ref, ek_ref, iek_ref, iev_ref):
    ek = ek_ref[...]
    norm = jnp.sqrt(jnp.sum(ek * ek, axis=1, keepdims=True))
    kn = ek / jnp.maximum(norm, 1e-12)
    s = lax.dot_general(
        xq_ref[...], kn, (((1,), (1,)), ((), ())),
        preferred_element_type=jnp.float32,
    )
    m = jnp.max(s, axis=1, keepdims=True)
    col = lax.broadcasted_iota(jnp.int32, s.shape, 1)
    idx = jnp.min(jnp.where(s >= m, col, jnp.int32(2**30)), axis=1)
    iek_ref[...] = 2 * idx
    iev_ref[...] = 2 * idx + 1


def _route(x_querry, e_k):
    return pl.pallas_call(
        _route_body,
        out_shape=(
            jax.ShapeDtypeStruct((_B,), jnp.int32),
            jax.ShapeDtypeStruct((_B,), jnp.int32),
        ),
    )(x_querry, e_k)


def _gather_body(table_hbm, iek_hbm, iev_hbm, ek_out, ev_out, idx_v, rows_v, sem):
    wid = lax.axis_index("s") * _NC + lax.axis_index("c")
    base = wid * _BPW
    pltpu.sync_copy(iek_hbm.at[pl.ds(base, _BPW)], idx_v)
    pltpu.async_copy(table_hbm.at[idx_v], rows_v, sem).wait()
    pltpu.sync_copy(rows_v, ek_out.at[pl.ds(base, _BPW)])
    pltpu.sync_copy(iev_hbm.at[pl.ds(base, _BPW)], idx_v)
    pltpu.async_copy(table_hbm.at[idx_v], rows_v, sem).wait()
    pltpu.sync_copy(rows_v, ev_out.at[pl.ds(base, _BPW)])


@functools.cache
def _make_gather():
    mesh = plsc.VectorSubcoreMesh(
        core_axis_name="c", subcore_axis_name="s", num_cores=_NC, num_subcores=_NS
    )
    return pl.kernel(
        _gather_body,
        out_type=(
            jax.ShapeDtypeStruct((_B, _ROW), jnp.float32),
            jax.ShapeDtypeStruct((_B, _ROW), jnp.float32),
        ),
        mesh=mesh,
        scratch_types=[
            pltpu.VMEM((_BPW,), jnp.int32),
            pltpu.VMEM((_BPW, _ROW), jnp.float32),
            pltpu.SemaphoreType.DMA,
        ],
    )


def kernel(x_querry, l, x_block, e_p, e_k):
    del l
    iek, iev = _route(x_querry, e_k)
    table = e_p.reshape(2 * _E_POOL, _ROW)
    ek_flat, ev_flat = _make_gather()(table, iek, iev)
    Ek = ek_flat.reshape(_B, _HALF, _EMB_D)
    Ev = ev_flat.reshape(_B, _HALF, _EMB_D)
    return (Ek, Ev, x_block)


# speedup vs baseline: 1.3057x; 1.3057x over previous
import functools

import jax
import jax.numpy as jnp
from jax import lax
from jax.experimental import pallas as pl
from jax.experimental.pallas import tpu as pltpu
from jax.experimental.pallas import tpu_sc as plsc

_EMB_D = 768
_E_POOL = 100
_E_P_LEN = 40
_B = 256
_HALF = _E_P_LEN // 2
_QB = 8
_STEPS = _B // _QB


def _route_body(xq_ref, ek_ref, idx_ref):
    ek = ek_ref[...]
    norm = jnp.sqrt(jnp.sum(ek * ek, axis=1, keepdims=True))
    kn = ek / jnp.maximum(norm, 1e-12)
    s = lax.dot_general(
        xq_ref[...], kn, (((1,), (1,)), ((), ())),
        preferred_element_type=jnp.float32,
    )
    m = jnp.max(s, axis=1, keepdims=True)
    col = lax.broadcasted_iota(jnp.int32, s.shape, 1)
    idx_ref[...] = jnp.min(jnp.where(s >= m, col, jnp.int32(2**30)), axis=1)


def _route(x_querry, e_k):
    return pl.pallas_call(
        _route_body,
        out_shape=jax.ShapeDtypeStruct((_B,), jnp.int32),
    )(x_querry, e_k)


def _gather_body(idx_ref, *refs):
    ep_refs = refs[:_QB]
    ek_ref, ev_ref = refs[_QB], refs[_QB + 1]
    for j in range(_QB):
        ek_ref[j] = ep_refs[j][0, :_HALF]
        ev_ref[j] = ep_refs[j][0, _HALF:]


def _gather(e_p, idx):
    ep_spec = [
        pl.BlockSpec(
            (1, _E_P_LEN, _EMB_D),
            functools.partial(lambda j, b, idx_ref: (idx_ref[_QB * b + j], 0, 0), j),
        )
        for j in range(_QB)
    ]
    out_spec = pl.BlockSpec((_QB, _HALF, _EMB_D), lambda b, idx_ref: (b, 0, 0))
    return pl.pallas_call(
        _gather_body,
        grid_spec=pltpu.PrefetchScalarGridSpec(
            num_scalar_prefetch=1,
            grid=(_STEPS,),
            in_specs=ep_spec,
            out_specs=[out_spec, out_spec],
        ),
        out_shape=[
            jax.ShapeDtypeStruct((_B, _HALF, _EMB_D), jnp.float32),
            jax.ShapeDtypeStruct((_B, _HALF, _EMB_D), jnp.float32),
        ],
        compiler_params=pltpu.CompilerParams(
            dimension_semantics=("arbitrary",),
        ),
    )(idx, *([e_p] * _QB))


def kernel(x_querry, l, x_block, e_p, e_k):
    del l
    idx = _route(x_querry, e_k)
    Ek, Ev = _gather(e_p, idx)
    return (Ek, Ev, x_block)


# speedup vs baseline: 1.5571x; 1.1925x over previous
import functools

import jax
import jax.numpy as jnp
from jax import lax
from jax.experimental import pallas as pl
from jax.experimental.pallas import tpu as pltpu
from jax.experimental.pallas import tpu_sc as plsc

_EMB_D = 768
_E_POOL = 100
_E_P_LEN = 40
_B = 256
_HALF = _E_P_LEN // 2
_QB = 8
_STEPS = _B // _QB


def _route_body(xq_ref, ek_ref, idx_ref):
    ek = ek_ref[...]
    norm = jnp.sqrt(jnp.sum(ek * ek, axis=1, keepdims=True))
    kn = ek / jnp.maximum(norm, 1e-12)
    s = lax.dot_general(
        xq_ref[...], kn, (((1,), (1,)), ((), ())),
        preferred_element_type=jnp.float32,
    )
    m = jnp.max(s, axis=1, keepdims=True)
    col = lax.broadcasted_iota(jnp.int32, s.shape, 1)
    idx_ref[...] = jnp.min(jnp.where(s >= m, col, jnp.int32(2**30)), axis=1)


def _route(x_querry, e_k):
    return pl.pallas_call(
        _route_body,
        out_shape=jax.ShapeDtypeStruct((_B,), jnp.int32),
    )(x_querry, e_k)


def _gather_body(idx_ref, *refs):
    ep_refs = refs[:_QB]
    ek_ref, ev_ref = refs[_QB], refs[_QB + 1]
    stacked = jnp.concatenate([r[...] for r in ep_refs], axis=0)
    swapped = jnp.swapaxes(stacked, 0, 1)
    ek_ref[...] = swapped[:_HALF]
    ev_ref[...] = swapped[_HALF:]


def _gather(e_p, idx):
    ep_spec = [
        pl.BlockSpec(
            (1, _E_P_LEN, _EMB_D),
            functools.partial(lambda j, b, idx_ref: (idx_ref[_QB * b + j], 0, 0), j),
        )
        for j in range(_QB)
    ]
    out_spec = pl.BlockSpec((_HALF, _QB, _EMB_D), lambda b, idx_ref: (0, b, 0))
    return pl.pallas_call(
        _gather_body,
        grid_spec=pltpu.PrefetchScalarGridSpec(
            num_scalar_prefetch=1,
            grid=(_STEPS,),
            in_specs=ep_spec,
            out_specs=[out_spec, out_spec],
        ),
        out_shape=[
            jax.ShapeDtypeStruct((_HALF, _B, _EMB_D), jnp.float32),
            jax.ShapeDtypeStruct((_HALF, _B, _EMB_D), jnp.float32),
        ],
        compiler_params=pltpu.CompilerParams(
            dimension_semantics=("arbitrary",),
        ),
    )(idx, *([e_p] * _QB))


def kernel(x_querry, l, x_block, e_p, e_k):
    del l
    idx = _route(x_querry, e_k)
    ek_t, ev_t = _gather(e_p, idx)
    Ek = jnp.swapaxes(ek_t, 0, 1)
    Ev = jnp.swapaxes(ev_t, 0, 1)
    return (Ek, Ev, x_block)


# speedup vs baseline: 1.6567x; 1.0640x over previous
import functools

import jax
import jax.numpy as jnp
from jax import lax
from jax.experimental import pallas as pl
from jax.experimental.pallas import tpu as pltpu
from jax.experimental.pallas import tpu_sc as plsc

_EMB_D = 768
_E_POOL = 100
_E_P_LEN = 40
_B = 256
_HALF = _E_P_LEN // 2
_QB = 16
_STEPS = _B // _QB


def _route_body(xq_ref, ek_ref, idx_ref):
    ek = ek_ref[...]
    norm = jnp.sqrt(jnp.sum(ek * ek, axis=1, keepdims=True))
    kn = ek / jnp.maximum(norm, 1e-12)
    s = lax.dot_general(
        xq_ref[...], kn, (((1,), (1,)), ((), ())),
        preferred_element_type=jnp.float32,
    )
    m = jnp.max(s, axis=1, keepdims=True)
    col = lax.broadcasted_iota(jnp.int32, s.shape, 1)
    idx_ref[...] = jnp.min(jnp.where(s >= m, col, jnp.int32(2**30)), axis=1)


def _route(x_querry, e_k):
    return pl.pallas_call(
        _route_body,
        out_shape=jax.ShapeDtypeStruct((_B,), jnp.int32),
    )(x_querry, e_k)


def _gather_body(idx_ref, *refs):
    ep_refs = refs[:_QB]
    ek_ref, ev_ref = refs[_QB], refs[_QB + 1]
    stacked = jnp.concatenate([r[...] for r in ep_refs], axis=0)
    swapped = jnp.swapaxes(stacked, 0, 1)
    ek_ref[...] = swapped[:_HALF]
    ev_ref[...] = swapped[_HALF:]


def _gather(e_p, idx):
    ep_spec = [
        pl.BlockSpec(
            (1, _E_P_LEN, _EMB_D),
            functools.partial(lambda j, b, idx_ref: (idx_ref[_QB * b + j], 0, 0), j),
        )
        for j in range(_QB)
    ]
    out_spec = pl.BlockSpec((_HALF, _QB, _EMB_D), lambda b, idx_ref: (0, b, 0))
    return pl.pallas_call(
        _gather_body,
        grid_spec=pltpu.PrefetchScalarGridSpec(
            num_scalar_prefetch=1,
            grid=(_STEPS,),
            in_specs=ep_spec,
            out_specs=[out_spec, out_spec],
        ),
        out_shape=[
            jax.ShapeDtypeStruct((_HALF, _B, _EMB_D), jnp.float32),
            jax.ShapeDtypeStruct((_HALF, _B, _EMB_D), jnp.float32),
        ],
        compiler_params=pltpu.CompilerParams(
            dimension_semantics=("arbitrary",),
        ),
    )(idx, *([e_p] * _QB))


def kernel(x_querry, l, x_block, e_p, e_k):
    del l
    idx = _route(x_querry, e_k)
    ek_t, ev_t = _gather(e_p, idx)
    Ek = jnp.swapaxes(ek_t, 0, 1)
    Ev = jnp.swapaxes(ev_t, 0, 1)
    return (Ek, Ev, x_block)


# speedup vs baseline: 1.6980x; 1.0249x over previous
import functools

import jax
import jax.numpy as jnp
from jax import lax
from jax.experimental import pallas as pl
from jax.experimental.pallas import tpu as pltpu
from jax.experimental.pallas import tpu_sc as plsc

_EMB_D = 768
_E_POOL = 100
_E_P_LEN = 40
_B = 256
_HALF = _E_P_LEN // 2
_QB = 32
_STEPS = _B // _QB


def _route_body(xq_ref, ek_ref, idx_ref):
    ek = ek_ref[...]
    norm = jnp.sqrt(jnp.sum(ek * ek, axis=1, keepdims=True))
    kn = ek / jnp.maximum(norm, 1e-12)
    s = lax.dot_general(
        xq_ref[...], kn, (((1,), (1,)), ((), ())),
        preferred_element_type=jnp.float32,
    )
    m = jnp.max(s, axis=1, keepdims=True)
    col = lax.broadcasted_iota(jnp.int32, s.shape, 1)
    idx_ref[...] = jnp.min(jnp.where(s >= m, col, jnp.int32(2**30)), axis=1)


def _route(x_querry, e_k):
    return pl.pallas_call(
        _route_body,
        out_shape=jax.ShapeDtypeStruct((_B,), jnp.int32),
    )(x_querry, e_k)


def _gather_body(idx_ref, *refs):
    ep_refs = refs[:_QB]
    ek_ref, ev_ref = refs[_QB], refs[_QB + 1]
    stacked = jnp.concatenate([r[...] for r in ep_refs], axis=0)
    swapped = jnp.swapaxes(stacked, 0, 1)
    ek_ref[...] = swapped[:_HALF]
    ev_ref[...] = swapped[_HALF:]


def _gather(e_p, idx):
    ep_spec = [
        pl.BlockSpec(
            (1, _E_P_LEN, _EMB_D),
            functools.partial(lambda j, b, idx_ref: (idx_ref[_QB * b + j], 0, 0), j),
        )
        for j in range(_QB)
    ]
    out_spec = pl.BlockSpec((_HALF, _QB, _EMB_D), lambda b, idx_ref: (0, b, 0))
    return pl.pallas_call(
        _gather_body,
        grid_spec=pltpu.PrefetchScalarGridSpec(
            num_scalar_prefetch=1,
            grid=(_STEPS,),
            in_specs=ep_spec,
            out_specs=[out_spec, out_spec],
        ),
        out_shape=[
            jax.ShapeDtypeStruct((_HALF, _B, _EMB_D), jnp.float32),
            jax.ShapeDtypeStruct((_HALF, _B, _EMB_D), jnp.float32),
        ],
        compiler_params=pltpu.CompilerParams(
            dimension_semantics=("arbitrary",),
        ),
    )(idx, *([e_p] * _QB))


def kernel(x_querry, l, x_block, e_p, e_k):
    del l
    idx = _route(x_querry, e_k)
    ek_t, ev_t = _gather(e_p, idx)
    Ek = jnp.swapaxes(ek_t, 0, 1)
    Ev = jnp.swapaxes(ev_t, 0, 1)
    return (Ek, Ev, x_block)


# speedup vs baseline: 1.7006x; 1.0015x over previous
import functools

import jax
import jax.numpy as jnp
from jax import lax
from jax.experimental import pallas as pl
from jax.experimental.pallas import tpu as pltpu
from jax.experimental.pallas import tpu_sc as plsc

_EMB_D = 768
_E_POOL = 100
_E_P_LEN = 40
_B = 256
_HALF = _E_P_LEN // 2
_QB = 64
_STEPS = _B // _QB


def _route_body(xq_ref, ek_ref, idx_ref):
    ek = ek_ref[...]
    norm = jnp.sqrt(jnp.sum(ek * ek, axis=1, keepdims=True))
    kn = ek / jnp.maximum(norm, 1e-12)
    s = lax.dot_general(
        xq_ref[...], kn, (((1,), (1,)), ((), ())),
        preferred_element_type=jnp.float32,
    )
    m = jnp.max(s, axis=1, keepdims=True)
    col = lax.broadcasted_iota(jnp.int32, s.shape, 1)
    idx_ref[...] = jnp.min(jnp.where(s >= m, col, jnp.int32(2**30)), axis=1)


def _route(x_querry, e_k):
    return pl.pallas_call(
        _route_body,
        out_shape=jax.ShapeDtypeStruct((_B,), jnp.int32),
    )(x_querry, e_k)


def _gather_body(idx_ref, *refs):
    ep_refs = refs[:_QB]
    ek_ref, ev_ref = refs[_QB], refs[_QB + 1]
    stacked = jnp.concatenate([r[...] for r in ep_refs], axis=0)
    swapped = jnp.swapaxes(stacked, 0, 1)
    ek_ref[...] = swapped[:_HALF]
    ev_ref[...] = swapped[_HALF:]


def _gather(e_p, idx):
    ep_spec = [
        pl.BlockSpec(
            (1, _E_P_LEN, _EMB_D),
            functools.partial(lambda j, b, idx_ref: (idx_ref[_QB * b + j], 0, 0), j),
        )
        for j in range(_QB)
    ]
    out_spec = pl.BlockSpec((_HALF, _QB, _EMB_D), lambda b, idx_ref: (0, b, 0))
    return pl.pallas_call(
        _gather_body,
        grid_spec=pltpu.PrefetchScalarGridSpec(
            num_scalar_prefetch=1,
            grid=(_STEPS,),
            in_specs=ep_spec,
            out_specs=[out_spec, out_spec],
        ),
        out_shape=[
            jax.ShapeDtypeStruct((_HALF, _B, _EMB_D), jnp.float32),
            jax.ShapeDtypeStruct((_HALF, _B, _EMB_D), jnp.float32),
        ],
        compiler_params=pltpu.CompilerParams(
            dimension_semantics=("arbitrary",),
        ),
    )(idx, *([e_p] * _QB))


def kernel(x_querry, l, x_block, e_p, e_k):
    del l
    idx = _route(x_querry, e_k)
    ek_t, ev_t = _gather(e_p, idx)
    Ek = jnp.swapaxes(ek_t, 0, 1)
    Ev = jnp.swapaxes(ev_t, 0, 1)
    return (Ek, Ev, x_block)
